# baseline (device time: 220280 ns/iter reference)
import jax
import jax.numpy as jnp
from jax import lax
from jax.experimental import pallas as pl
from jax.experimental.pallas import tpu as pltpu

N_DEV = 16
B = 2
S_LOC = 256
HQ = 4
DH = 64
BLK = 64
R = S_LOC // BLK
D_MODEL = 512
D_QK = HQ * DH
ROWS = B * S_LOC
KV_ROWS = 2 * ROWS


def kernel(x, Wq, K_ext, V_ext, Wo):
    x2 = x.reshape(ROWS, D_MODEL)
    k2 = K_ext.reshape(ROWS, D_QK)
    v2 = V_ext.reshape(ROWS, D_QK)

    def body(x_ref, wq_ref, k_ref, v_ref, wo_ref, out_ref,
             gkv, ctx_ref, send_sems, recv_sems):
        my = lax.axis_index("i")
        left = lax.rem(my + N_DEV - 1, N_DEV)
        right = lax.rem(my + 1, N_DEV)

        barrier = pltpu.get_barrier_semaphore()
        pl.semaphore_signal(barrier, inc=1, device_id=(left,),
                            device_id_type=pl.DeviceIdType.MESH)
        pl.semaphore_signal(barrier, inc=1, device_id=(right,),
                            device_id_type=pl.DeviceIdType.MESH)
        pl.semaphore_wait(barrier, 2)

        gkv[0, 0:ROWS, :] = k_ref[:, :]
        gkv[0, ROWS:KV_ROWS, :] = v_ref[:, :]

        for h in range(N_DEV - 1):
            rdma = pltpu.make_async_remote_copy(
                src_ref=gkv.at[h],
                dst_ref=gkv.at[h + 1],
                send_sem=send_sems.at[h],
                recv_sem=recv_sems.at[h],
                device_id=(right,),
                device_id_type=pl.DeviceIdType.MESH,
            )
            rdma.start()
            rdma.wait()

        q_all = jnp.dot(x_ref[:, :], wq_ref[:, :],
                        preferred_element_type=jnp.float32)

        for b in range(B):
            for r in range(R):
                off = b * S_LOC + r * BLK
                kr = gkv[:, off:off + BLK, :].reshape(N_DEV * BLK, D_QK)
                vr = gkv[:, ROWS + off:ROWS + off + BLK, :].reshape(
                    N_DEV * BLK, D_QK)
                qr = q_all[off:off + BLK, :]
                for hh in range(HQ):
                    c0 = hh * DH
                    qh = qr[:, c0:c0 + DH]
                    kh = kr[:, c0:c0 + DH]
                    s = lax.dot_general(
                        qh, kh, (((1,), (1,)), ((), ())),
                        preferred_element_type=jnp.float32) * 0.125
                    m = jnp.max(s, axis=1, keepdims=True)
                    w = jnp.exp(s - m)
                    w = w / jnp.sum(w, axis=1, keepdims=True)
                    ctx_ref[off:off + BLK, c0:c0 + DH] = jnp.dot(
                        w, vr[:, c0:c0 + DH],
                        preferred_element_type=jnp.float32)

        out_ref[:, :] = jnp.dot(ctx_ref[:, :], wo_ref[:, :],
                                preferred_element_type=jnp.float32)

    out = pl.pallas_call(
        body,
        out_shape=jax.ShapeDtypeStruct((ROWS, D_MODEL), jnp.float32),
        in_specs=[pl.BlockSpec(memory_space=pltpu.VMEM)] * 5,
        out_specs=pl.BlockSpec(memory_space=pltpu.VMEM),
        scratch_shapes=[
            pltpu.VMEM((N_DEV, KV_ROWS, D_QK), jnp.float32),
            pltpu.VMEM((ROWS, D_QK), jnp.float32),
            pltpu.SemaphoreType.DMA((N_DEV - 1,)),
            pltpu.SemaphoreType.DMA((N_DEV - 1,)),
        ],
        compiler_params=pltpu.CompilerParams(collective_id=0),
    )(x2, Wq, k2, v2, Wo)
    return out.reshape(B, S_LOC, D_MODEL)


# device time: 141096 ns/iter; 1.5612x vs baseline; 1.5612x over previous
import jax
import jax.numpy as jnp
from jax import lax
from jax.experimental import pallas as pl
from jax.experimental.pallas import tpu as pltpu

N_DEV = 16
B = 2
S_LOC = 256
HQ = 4
DH = 64
BLK = 64
R = S_LOC // BLK
D_MODEL = 512
D_QK = HQ * DH
ROWS = B * S_LOC
KV_ROWS = 2 * ROWS


def kernel(x, Wq, K_ext, V_ext, Wo):
    x2 = x.reshape(ROWS, D_MODEL)
    k2 = K_ext.reshape(ROWS, D_QK)
    v2 = V_ext.reshape(ROWS, D_QK)

    def body(x_ref, wq_ref, k_ref, v_ref, wo_ref, out_ref,
             gkv, ctx_ref, send_sems, recv_sems):
        my = lax.axis_index("i")
        left = lax.rem(my + N_DEV - 1, N_DEV)
        right = lax.rem(my + 1, N_DEV)

        barrier = pltpu.get_barrier_semaphore()
        pl.semaphore_signal(barrier, inc=1, device_id=(left,),
                            device_id_type=pl.DeviceIdType.MESH)
        pl.semaphore_signal(barrier, inc=1, device_id=(right,),
                            device_id_type=pl.DeviceIdType.MESH)
        pl.semaphore_wait(barrier, 2)

        gkv[0, 0:ROWS, :] = k_ref[:, :]
        gkv[0, ROWS:KV_ROWS, :] = v_ref[:, :]

        N_CW = 8
        N_CCW = 7
        for t in range(N_CW):
            rd_cw = pltpu.make_async_remote_copy(
                src_ref=gkv.at[t],
                dst_ref=gkv.at[t + 1],
                send_sem=send_sems.at[t],
                recv_sem=recv_sems.at[t],
                device_id=(right,),
                device_id_type=pl.DeviceIdType.MESH,
            )
            rd_cw.start()
            rd_ccw = None
            if t < N_CCW:
                src_slot = 0 if t == 0 else 8 + t
                rd_ccw = pltpu.make_async_remote_copy(
                    src_ref=gkv.at[src_slot],
                    dst_ref=gkv.at[9 + t],
                    send_sem=send_sems.at[N_CW + t],
                    recv_sem=recv_sems.at[N_CW + t],
                    device_id=(left,),
                    device_id_type=pl.DeviceIdType.MESH,
                )
                rd_ccw.start()
            rd_cw.wait()
            if rd_ccw is not None:
                rd_ccw.wait()

        q_all = jnp.dot(x_ref[:, :], wq_ref[:, :],
                        preferred_element_type=jnp.float32)

        for b in range(B):
            for r in range(R):
                off = b * S_LOC + r * BLK
                kr = gkv[:, off:off + BLK, :].reshape(N_DEV * BLK, D_QK)
                vr = gkv[:, ROWS + off:ROWS + off + BLK, :].reshape(
                    N_DEV * BLK, D_QK)
                qr = q_all[off:off + BLK, :]
                for hh in range(HQ):
                    c0 = hh * DH
                    qh = qr[:, c0:c0 + DH]
                    kh = kr[:, c0:c0 + DH]
                    s = lax.dot_general(
                        qh, kh, (((1,), (1,)), ((), ())),
                        preferred_element_type=jnp.float32) * 0.125
                    m = jnp.max(s, axis=1, keepdims=True)
                    w = jnp.exp(s - m)
                    w = w / jnp.sum(w, axis=1, keepdims=True)
                    ctx_ref[off:off + BLK, c0:c0 + DH] = jnp.dot(
                        w, vr[:, c0:c0 + DH],
                        preferred_element_type=jnp.float32)

        out_ref[:, :] = jnp.dot(ctx_ref[:, :], wo_ref[:, :],
                                preferred_element_type=jnp.float32)

    out = pl.pallas_call(
        body,
        out_shape=jax.ShapeDtypeStruct((ROWS, D_MODEL), jnp.float32),
        in_specs=[pl.BlockSpec(memory_space=pltpu.VMEM)] * 5,
        out_specs=pl.BlockSpec(memory_space=pltpu.VMEM),
        scratch_shapes=[
            pltpu.VMEM((N_DEV, KV_ROWS, D_QK), jnp.float32),
            pltpu.VMEM((ROWS, D_QK), jnp.float32),
            pltpu.SemaphoreType.DMA((N_DEV - 1,)),
            pltpu.SemaphoreType.DMA((N_DEV - 1,)),
        ],
        compiler_params=pltpu.CompilerParams(collective_id=0),
    )(x2, Wq, k2, v2, Wo)
    return out.reshape(B, S_LOC, D_MODEL)


# device time: 130603 ns/iter; 1.6866x vs baseline; 1.0803x over previous
import jax
import jax.numpy as jnp
from jax import lax
from jax.experimental import pallas as pl
from jax.experimental.pallas import tpu as pltpu

N_DEV = 16
B = 2
S_LOC = 256
HQ = 4
DH = 64
BLK = 64
R = S_LOC // BLK
D_MODEL = 512
D_QK = HQ * DH
ROWS = B * S_LOC
KV_ROWS = 2 * ROWS

RING = [0, 1, 5, 9, 13, 14, 10, 6, 2, 3, 7, 11, 15, 12, 8, 4]
_POS = {m: p for p, m in enumerate(RING)}
NXT = [RING[(_POS[m] + 1) % N_DEV] for m in range(N_DEV)]
PRV = [RING[(_POS[m] - 1) % N_DEV] for m in range(N_DEV)]


def kernel(x, Wq, K_ext, V_ext, Wo):
    x2 = x.reshape(ROWS, D_MODEL)
    k2 = K_ext.reshape(ROWS, D_QK)
    v2 = V_ext.reshape(ROWS, D_QK)

    my = lax.axis_index("i")
    nxt = jnp.asarray(NXT, jnp.int32)[my].reshape(1)
    prv = jnp.asarray(PRV, jnp.int32)[my].reshape(1)

    def body(nxt_ref, prv_ref, x_ref, wq_ref, k_ref, v_ref, wo_ref, out_ref,
             gkv, ctx_ref, send_sems, recv_sems):
        left = prv_ref[0]
        right = nxt_ref[0]

        barrier = pltpu.get_barrier_semaphore()
        pl.semaphore_signal(barrier, inc=1, device_id=(left,),
                            device_id_type=pl.DeviceIdType.MESH)
        pl.semaphore_signal(barrier, inc=1, device_id=(right,),
                            device_id_type=pl.DeviceIdType.MESH)
        pl.semaphore_wait(barrier, 2)

        gkv[0, 0:ROWS, :] = k_ref[:, :]
        gkv[0, ROWS:KV_ROWS, :] = v_ref[:, :]

        N_CW = 8
        N_CCW = 7

        def mk_cw(t):
            return pltpu.make_async_remote_copy(
                src_ref=gkv.at[t],
                dst_ref=gkv.at[t + 1],
                send_sem=send_sems.at[t],
                recv_sem=recv_sems.at[t],
                device_id=(right,),
                device_id_type=pl.DeviceIdType.MESH,
            )

        def mk_ccw(t):
            return pltpu.make_async_remote_copy(
                src_ref=gkv.at[0 if t == 0 else 8 + t],
                dst_ref=gkv.at[9 + t],
                send_sem=send_sems.at[N_CW + t],
                recv_sem=recv_sems.at[N_CW + t],
                device_id=(left,),
                device_id_type=pl.DeviceIdType.MESH,
            )

        cw = [mk_cw(0)]
        ccw = [mk_ccw(0)]
        cw[0].start()
        ccw[0].start()

        q_all = jnp.dot(x_ref[:, :], wq_ref[:, :],
                        preferred_element_type=jnp.float32)

        for t in range(N_CW):
            cw[t].wait_recv()
            if t + 1 < N_CW:
                cw.append(mk_cw(t + 1))
                cw[t + 1].start()
            if t < N_CCW:
                ccw[t].wait_recv()
                if t + 1 < N_CCW:
                    ccw.append(mk_ccw(t + 1))
                    ccw[t + 1].start()

        for rd in cw + ccw:
            rd.wait_send()

        for b in range(B):
            for r in range(R):
                off = b * S_LOC + r * BLK
                kr = gkv[:, off:off + BLK, :].reshape(N_DEV * BLK, D_QK)
                vr = gkv[:, ROWS + off:ROWS + off + BLK, :].reshape(
                    N_DEV * BLK, D_QK)
                qr = q_all[off:off + BLK, :]
                for hh in range(HQ):
                    c0 = hh * DH
                    qh = qr[:, c0:c0 + DH]
                    kh = kr[:, c0:c0 + DH]
                    s = lax.dot_general(
                        qh, kh, (((1,), (1,)), ((), ())),
                        preferred_element_type=jnp.float32) * 0.125
                    m = jnp.max(s, axis=1, keepdims=True)
                    w = jnp.exp(s - m)
                    w = w / jnp.sum(w, axis=1, keepdims=True)
                    ctx_ref[off:off + BLK, c0:c0 + DH] = jnp.dot(
                        w, vr[:, c0:c0 + DH],
                        preferred_element_type=jnp.float32)

        out_ref[:, :] = jnp.dot(ctx_ref[:, :], wo_ref[:, :],
                                preferred_element_type=jnp.float32)

    out = pl.pallas_call(
        body,
        out_shape=jax.ShapeDtypeStruct((ROWS, D_MODEL), jnp.float32),
        in_specs=[pl.BlockSpec(memory_space=pltpu.SMEM)] * 2
        + [pl.BlockSpec(memory_space=pltpu.VMEM)] * 5,
        out_specs=pl.BlockSpec(memory_space=pltpu.VMEM),
        scratch_shapes=[
            pltpu.VMEM((N_DEV, KV_ROWS, D_QK), jnp.float32),
            pltpu.VMEM((ROWS, D_QK), jnp.float32),
            pltpu.SemaphoreType.DMA((N_DEV - 1,)),
            pltpu.SemaphoreType.DMA((N_DEV - 1,)),
        ],
        compiler_params=pltpu.CompilerParams(collective_id=0),
    )(nxt, prv, x2, Wq, k2, v2, Wo)
    return out.reshape(B, S_LOC, D_MODEL)


# device time: 85812 ns/iter; 2.5670x vs baseline; 1.5220x over previous
import jax
import jax.numpy as jnp
from jax import lax
from jax.experimental import pallas as pl
from jax.experimental.pallas import tpu as pltpu

N_DEV = 16
B = 2
S_LOC = 256
HQ = 4
DH = 64
BLK = 64
R = S_LOC // BLK
D_MODEL = 512
D_QK = HQ * DH
ROWS = B * S_LOC
KV_ROWS = 2 * ROWS

RING = [0, 1, 5, 9, 13, 14, 10, 6, 2, 3, 7, 11, 15, 12, 8, 4]
_POS = {m: p for p, m in enumerate(RING)}
NXT = [RING[(_POS[m] + 1) % N_DEV] for m in range(N_DEV)]
PRV = [RING[(_POS[m] - 1) % N_DEV] for m in range(N_DEV)]


def kernel(x, Wq, K_ext, V_ext, Wo):
    x2 = x.reshape(ROWS, D_MODEL)
    k2 = K_ext.reshape(ROWS, D_QK).astype(jnp.bfloat16)
    v2 = V_ext.reshape(ROWS, D_QK).astype(jnp.bfloat16)

    my = lax.axis_index("i")
    nxt = jnp.asarray(NXT, jnp.int32)[my].reshape(1)
    prv = jnp.asarray(PRV, jnp.int32)[my].reshape(1)

    def body(nxt_ref, prv_ref, x_ref, wq_ref, k_ref, v_ref, wo_ref, out_ref,
             gkv, ctx_ref, send_sems, recv_sems):
        left = prv_ref[0]
        right = nxt_ref[0]

        barrier = pltpu.get_barrier_semaphore()
        pl.semaphore_signal(barrier, inc=1, device_id=(left,),
                            device_id_type=pl.DeviceIdType.MESH)
        pl.semaphore_signal(barrier, inc=1, device_id=(right,),
                            device_id_type=pl.DeviceIdType.MESH)
        pl.semaphore_wait(barrier, 2)

        gkv[0, 0:ROWS, :] = k_ref[:, :]
        gkv[0, ROWS:KV_ROWS, :] = v_ref[:, :]

        N_CW = 8
        N_CCW = 7

        def mk_cw(t):
            return pltpu.make_async_remote_copy(
                src_ref=gkv.at[t],
                dst_ref=gkv.at[t + 1],
                send_sem=send_sems.at[t],
                recv_sem=recv_sems.at[t],
                device_id=(right,),
                device_id_type=pl.DeviceIdType.MESH,
            )

        def mk_ccw(t):
            return pltpu.make_async_remote_copy(
                src_ref=gkv.at[0 if t == 0 else 8 + t],
                dst_ref=gkv.at[9 + t],
                send_sem=send_sems.at[N_CW + t],
                recv_sem=recv_sems.at[N_CW + t],
                device_id=(left,),
                device_id_type=pl.DeviceIdType.MESH,
            )

        cw = [mk_cw(0)]
        ccw = [mk_ccw(0)]
        cw[0].start()
        ccw[0].start()

        q_all = jnp.dot(x_ref[:, :], wq_ref[:, :],
                        preferred_element_type=jnp.float32)

        for t in range(N_CW):
            cw[t].wait_recv()
            if t + 1 < N_CW:
                cw.append(mk_cw(t + 1))
                cw[t + 1].start()
            if t < N_CCW:
                ccw[t].wait_recv()
                if t + 1 < N_CCW:
                    ccw.append(mk_ccw(t + 1))
                    ccw[t + 1].start()

        for rd in cw + ccw:
            rd.wait_send()

        for b in range(B):
            for r in range(R):
                off = b * S_LOC + r * BLK
                kr = gkv[:, off:off + BLK, :].reshape(N_DEV * BLK, D_QK)
                vr = gkv[:, ROWS + off:ROWS + off + BLK, :].reshape(
                    N_DEV * BLK, D_QK)
                qr = q_all[off:off + BLK, :].astype(jnp.bfloat16)
                for hh in range(HQ):
                    c0 = hh * DH
                    qh = qr[:, c0:c0 + DH]
                    kh = kr[:, c0:c0 + DH]
                    s = lax.dot_general(
                        qh, kh, (((1,), (1,)), ((), ())),
                        preferred_element_type=jnp.float32) * 0.125
                    m = jnp.max(s, axis=1, keepdims=True)
                    w = jnp.exp(s - m)
                    w = (w / jnp.sum(w, axis=1, keepdims=True)).astype(
                        jnp.bfloat16)
                    ctx_ref[off:off + BLK, c0:c0 + DH] = jnp.dot(
                        w, vr[:, c0:c0 + DH],
                        preferred_element_type=jnp.float32)

        out_ref[:, :] = jnp.dot(ctx_ref[:, :], wo_ref[:, :],
                                preferred_element_type=jnp.float32)

    out = pl.pallas_call(
        body,
        out_shape=jax.ShapeDtypeStruct((ROWS, D_MODEL), jnp.float32),
        in_specs=[pl.BlockSpec(memory_space=pltpu.SMEM)] * 2
        + [pl.BlockSpec(memory_space=pltpu.VMEM)] * 5,
        out_specs=pl.BlockSpec(memory_space=pltpu.VMEM),
        scratch_shapes=[
            pltpu.VMEM((N_DEV, KV_ROWS, D_QK), jnp.bfloat16),
            pltpu.VMEM((ROWS, D_QK), jnp.float32),
            pltpu.SemaphoreType.DMA((N_DEV - 1,)),
            pltpu.SemaphoreType.DMA((N_DEV - 1,)),
        ],
        compiler_params=pltpu.CompilerParams(collective_id=0),
    )(nxt, prv, x2, Wq, k2, v2, Wo)
    return out.reshape(B, S_LOC, D_MODEL)


# device time: 83212 ns/iter; 2.6472x vs baseline; 1.0312x over previous
import jax
import jax.numpy as jnp
from jax import lax
from jax.experimental import pallas as pl
from jax.experimental.pallas import tpu as pltpu

N_DEV = 16
B = 2
S_LOC = 256
HQ = 4
DH = 64
BLK = 64
R = S_LOC // BLK
D_MODEL = 512
D_QK = HQ * DH
ROWS = B * S_LOC
KV_ROWS = 2 * ROWS

RING = [0, 1, 5, 9, 13, 14, 10, 6, 2, 3, 7, 11, 15, 12, 8, 4]
_POS = {m: p for p, m in enumerate(RING)}
NXT = [RING[(_POS[m] + 1) % N_DEV] for m in range(N_DEV)]
PRV = [RING[(_POS[m] - 1) % N_DEV] for m in range(N_DEV)]


def kernel(x, Wq, K_ext, V_ext, Wo):
    x2 = x.reshape(ROWS, D_MODEL)
    k2 = K_ext.reshape(ROWS, D_QK)
    v2 = V_ext.reshape(ROWS, D_QK)

    my = lax.axis_index("i")
    nxt = jnp.asarray(NXT, jnp.int32)[my].reshape(1)
    prv = jnp.asarray(PRV, jnp.int32)[my].reshape(1)

    def body(nxt_ref, prv_ref, x_ref, wq_ref, k_ref, v_ref, wo_ref, out_ref,
             gkv, ctx_ref, send_sems, recv_sems):
        left = prv_ref[0]
        right = nxt_ref[0]

        barrier = pltpu.get_barrier_semaphore()
        pl.semaphore_signal(barrier, inc=1, device_id=(left,),
                            device_id_type=pl.DeviceIdType.MESH)
        pl.semaphore_signal(barrier, inc=1, device_id=(right,),
                            device_id_type=pl.DeviceIdType.MESH)
        pl.semaphore_wait(barrier, 2)

        gkv[0, 0:ROWS, :] = k_ref[:, :].astype(jnp.bfloat16)
        gkv[0, ROWS:KV_ROWS, :] = v_ref[:, :].astype(jnp.bfloat16)

        N_RND = 8

        def mk_cw(t):
            if t == N_RND - 1:
                return pltpu.make_async_remote_copy(
                    src_ref=gkv.at[7, pl.ds(0, ROWS), :],
                    dst_ref=gkv.at[8, pl.ds(0, ROWS), :],
                    send_sem=send_sems.at[t],
                    recv_sem=recv_sems.at[t],
                    device_id=(right,),
                    device_id_type=pl.DeviceIdType.MESH,
                )
            return pltpu.make_async_remote_copy(
                src_ref=gkv.at[t],
                dst_ref=gkv.at[t + 1],
                send_sem=send_sems.at[t],
                recv_sem=recv_sems.at[t],
                device_id=(right,),
                device_id_type=pl.DeviceIdType.MESH,
            )

        def mk_ccw(t):
            if t == N_RND - 1:
                return pltpu.make_async_remote_copy(
                    src_ref=gkv.at[15, pl.ds(ROWS, ROWS), :],
                    dst_ref=gkv.at[8, pl.ds(ROWS, ROWS), :],
                    send_sem=send_sems.at[N_RND + t],
                    recv_sem=recv_sems.at[N_RND + t],
                    device_id=(left,),
                    device_id_type=pl.DeviceIdType.MESH,
                )
            return pltpu.make_async_remote_copy(
                src_ref=gkv.at[0 if t == 0 else 8 + t],
                dst_ref=gkv.at[9 + t],
                send_sem=send_sems.at[N_RND + t],
                recv_sem=recv_sems.at[N_RND + t],
                device_id=(left,),
                device_id_type=pl.DeviceIdType.MESH,
            )

        cw = [mk_cw(0)]
        ccw = [mk_ccw(0)]
        cw[0].start()
        ccw[0].start()

        q_all = jnp.dot(x_ref[:, :].astype(jnp.bfloat16),
                        wq_ref[:, :].astype(jnp.bfloat16),
                        preferred_element_type=jnp.float32)

        for t in range(N_RND):
            cw[t].wait_recv()
            if t + 1 < N_RND:
                cw.append(mk_cw(t + 1))
                cw[t + 1].start()
            ccw[t].wait_recv()
            if t + 1 < N_RND:
                ccw.append(mk_ccw(t + 1))
                ccw[t + 1].start()

        for rd in cw + ccw:
            rd.wait_send()

        for b in range(B):
            for r in range(R):
                off = b * S_LOC + r * BLK
                kr = gkv[:, off:off + BLK, :].reshape(N_DEV * BLK, D_QK)
                vr = gkv[:, ROWS + off:ROWS + off + BLK, :].reshape(
                    N_DEV * BLK, D_QK)
                qr = q_all[off:off + BLK, :].astype(jnp.bfloat16)
                for hh in range(HQ):
                    c0 = hh * DH
                    qh = qr[:, c0:c0 + DH]
                    kh = kr[:, c0:c0 + DH]
                    s = lax.dot_general(
                        qh, kh, (((1,), (1,)), ((), ())),
                        preferred_element_type=jnp.float32) * 0.125
                    m = jnp.max(s, axis=1, keepdims=True)
                    w = jnp.exp(s - m)
                    w = (w / jnp.sum(w, axis=1, keepdims=True)).astype(
                        jnp.bfloat16)
                    ctx_ref[off:off + BLK, c0:c0 + DH] = jnp.dot(
                        w, vr[:, c0:c0 + DH],
                        preferred_element_type=jnp.float32).astype(
                            jnp.bfloat16)

        out_ref[:, :] = jnp.dot(ctx_ref[:, :],
                                wo_ref[:, :].astype(jnp.bfloat16),
                                preferred_element_type=jnp.float32)

    out = pl.pallas_call(
        body,
        out_shape=jax.ShapeDtypeStruct((ROWS, D_MODEL), jnp.float32),
        in_specs=[pl.BlockSpec(memory_space=pltpu.SMEM)] * 2
        + [pl.BlockSpec(memory_space=pltpu.VMEM)] * 5,
        out_specs=pl.BlockSpec(memory_space=pltpu.VMEM),
        scratch_shapes=[
            pltpu.VMEM((N_DEV, KV_ROWS, D_QK), jnp.bfloat16),
            pltpu.VMEM((ROWS, D_QK), jnp.bfloat16),
            pltpu.SemaphoreType.DMA((16,)),
            pltpu.SemaphoreType.DMA((16,)),
        ],
        compiler_params=pltpu.CompilerParams(collective_id=0),
    )(nxt, prv, x2, Wq, k2, v2, Wo)
    return out.reshape(B, S_LOC, D_MODEL)
